# baseline (speedup 1.0000x reference)
import jax
import jax.numpy as jnp
from jax.experimental import pallas as pl
from jax.experimental.pallas import tpu as pltpu

S = 8192
D = 4096
E = 64
C = 128
T = 256
NBLK = S // T


def _route_kernel(x_ref, w_ref, idx_ref, loc_ref, gate_ref, laux_ref,
                  cnt_ref, me_ref):
    i = pl.program_id(0)

    @pl.when(i == 0)
    def _init():
        cnt_ref[...] = jnp.zeros_like(cnt_ref)
        me_ref[...] = jnp.zeros_like(me_ref)

    logits = jnp.dot(x_ref[...], w_ref[...],
                     preferred_element_type=jnp.float32)
    mx = jnp.max(logits, axis=1, keepdims=True)
    ex = jnp.exp(logits - mx)
    denom = jnp.sum(ex, axis=1, keepdims=True)
    gates = ex / denom

    gmax = jnp.max(gates, axis=1, keepdims=True)
    eiota = jax.lax.broadcasted_iota(jnp.int32, (T, E), 1)
    idx = jnp.min(jnp.where(gates == gmax, eiota, E), axis=1,
                  keepdims=True)
    maskf = (eiota == idx).astype(jnp.float32)

    r = jax.lax.broadcasted_iota(jnp.int32, (T, T), 0)
    c = jax.lax.broadcasted_iota(jnp.int32, (T, T), 1)
    tril = (c <= r).astype(jnp.bfloat16)
    counts = jnp.dot(tril, maskf.astype(jnp.bfloat16),
                     preferred_element_type=jnp.float32)

    loc = counts - 1.0 + cnt_ref[...]
    loc_s = jnp.sum(loc * maskf, axis=1, keepdims=True)
    keep = loc_s < float(C)
    loc_i = loc_s.astype(jnp.int32)
    idx_eff = jnp.where(keep, idx, E)

    idx_ref[...] = idx_eff
    loc_ref[...] = loc_i
    gate_ref[...] = gmax

    cnt_ref[...] = cnt_ref[...] + counts[T - 1:T, :]
    me_ref[...] = me_ref[...] + jnp.sum(gates, axis=0, keepdims=True)

    @pl.when(i == NBLK - 1)
    def _fini():
        scale = float(E) / (float(S) * float(S))
        laux_ref[0, 0] = jnp.sum(me_ref[...] * cnt_ref[...]) * scale


def _combine_kernel(idx_ref, loc_ref, gate_ref, comb_ref):
    pos3 = (idx_ref[...] * C + loc_ref[...]).reshape(T, 1, 1)
    gate3 = gate_ref[...].reshape(T, 1, 1)
    pe = (jax.lax.broadcasted_iota(jnp.int32, (1, E, C), 1) * C
          + jax.lax.broadcasted_iota(jnp.int32, (1, E, C), 2))
    comb_ref[...] = jnp.where(pe == pos3, gate3, 0.0)


@jax.jit
def kernel(x, W):
    idx_eff, loc_i, gate_s, laux = pl.pallas_call(
        _route_kernel,
        grid=(NBLK,),
        in_specs=[
            pl.BlockSpec((T, D), lambda i: (i, 0)),
            pl.BlockSpec((D, E), lambda i: (0, 0)),
        ],
        out_specs=[
            pl.BlockSpec((T, 1), lambda i: (i, 0)),
            pl.BlockSpec((T, 1), lambda i: (i, 0)),
            pl.BlockSpec((T, 1), lambda i: (i, 0)),
            pl.BlockSpec((1, 1), lambda i: (0, 0), memory_space=pltpu.SMEM),
        ],
        out_shape=[
            jax.ShapeDtypeStruct((S, 1), jnp.int32),
            jax.ShapeDtypeStruct((S, 1), jnp.int32),
            jax.ShapeDtypeStruct((S, 1), jnp.float32),
            jax.ShapeDtypeStruct((1, 1), jnp.float32),
        ],
        scratch_shapes=[
            pltpu.VMEM((1, E), jnp.float32),
            pltpu.VMEM((1, E), jnp.float32),
        ],
    )(x, W)

    combine = pl.pallas_call(
        _combine_kernel,
        grid=(NBLK,),
        in_specs=[
            pl.BlockSpec((T, 1), lambda i: (i, 0)),
            pl.BlockSpec((T, 1), lambda i: (i, 0)),
            pl.BlockSpec((T, 1), lambda i: (i, 0)),
        ],
        out_specs=pl.BlockSpec((T, E, C), lambda i: (i, 0, 0)),
        out_shape=jax.ShapeDtypeStruct((S, E, C), jnp.float32),
    )(idx_eff, loc_i, gate_s)

    l_aux = laux[0, 0]
    iv = idx_eff.reshape(S, 1, 1)
    lv = loc_i.reshape(S, 1, 1)
    dispatch = (iv == jnp.arange(E, dtype=jnp.int32).reshape(1, E, 1)) & \
               (lv == jnp.arange(C, dtype=jnp.int32).reshape(1, 1, C))
    return (l_aux, combine, dispatch)


# speedup vs baseline: 1.1258x; 1.1258x over previous
import jax
import jax.numpy as jnp
from jax.experimental import pallas as pl
from jax.experimental.pallas import tpu as pltpu

S = 8192
D = 4096
E = 64
C = 128
T = 512
NBLK = S // T


def _gate_kernel(x_ref, w_ref, comb_ref, idx_ref, loc_ref, laux_ref,
                 cnt_ref, me_ref):
    i = pl.program_id(0)

    @pl.when(i == 0)
    def _init():
        cnt_ref[...] = jnp.zeros_like(cnt_ref)
        me_ref[...] = jnp.zeros_like(me_ref)

    logits = jnp.dot(x_ref[...], w_ref[...],
                     preferred_element_type=jnp.float32)
    mx = jnp.max(logits, axis=1, keepdims=True)
    ex = jnp.exp(logits - mx)
    denom = jnp.sum(ex, axis=1, keepdims=True)
    gates = ex / denom

    gmax = jnp.max(gates, axis=1, keepdims=True)
    eiota = jax.lax.broadcasted_iota(jnp.int32, (T, E), 1)
    idx = jnp.min(jnp.where(gates == gmax, eiota, E), axis=1,
                  keepdims=True)
    maskf = (eiota == idx).astype(jnp.float32)

    r = jax.lax.broadcasted_iota(jnp.int32, (T, T), 0)
    c = jax.lax.broadcasted_iota(jnp.int32, (T, T), 1)
    tril = (c <= r).astype(jnp.bfloat16)
    counts = jnp.dot(tril, maskf.astype(jnp.bfloat16),
                     preferred_element_type=jnp.float32)

    loc = counts - 1.0 + cnt_ref[...]
    loc_s = jnp.sum(loc * maskf, axis=1, keepdims=True)
    keep = loc_s < float(C)
    loc_i = loc_s.astype(jnp.int32)
    idx_eff = jnp.where(keep, idx, E)

    pe = (jax.lax.broadcasted_iota(jnp.int32, (1, E, C), 1) * C
          + jax.lax.broadcasted_iota(jnp.int32, (1, E, C), 2))
    pos3 = (idx_eff * C + loc_i).reshape(T, 1, 1)
    comb_ref[...] = jnp.where(pe == pos3, gmax.reshape(T, 1, 1), 0.0)

    idx_ref[...] = idx_eff.reshape(1, 1, T)
    loc_ref[...] = loc_i.reshape(1, 1, T)

    cnt_ref[...] = cnt_ref[...] + counts[T - 1:T, :]
    me_ref[...] = me_ref[...] + jnp.sum(gates, axis=0, keepdims=True)

    @pl.when(i == NBLK - 1)
    def _fini():
        scale = float(E) / (float(S) * float(S))
        laux_ref[0, 0] = jnp.sum(me_ref[...] * cnt_ref[...]) * scale


@jax.jit
def kernel(x, W):
    combine, idx_eff, loc_i, laux = pl.pallas_call(
        _gate_kernel,
        grid=(NBLK,),
        in_specs=[
            pl.BlockSpec((T, D), lambda i: (i, 0)),
            pl.BlockSpec((D, E), lambda i: (0, 0)),
        ],
        out_specs=[
            pl.BlockSpec((T, E, C), lambda i: (i, 0, 0)),
            pl.BlockSpec((1, 1, T), lambda i: (i, 0, 0)),
            pl.BlockSpec((1, 1, T), lambda i: (i, 0, 0)),
            pl.BlockSpec((1, 1), lambda i: (0, 0), memory_space=pltpu.SMEM),
        ],
        out_shape=[
            jax.ShapeDtypeStruct((S, E, C), jnp.float32),
            jax.ShapeDtypeStruct((NBLK, 1, T), jnp.int32),
            jax.ShapeDtypeStruct((NBLK, 1, T), jnp.int32),
            jax.ShapeDtypeStruct((1, 1), jnp.float32),
        ],
        scratch_shapes=[
            pltpu.VMEM((1, E), jnp.float32),
            pltpu.VMEM((1, E), jnp.float32),
        ],
    )(x, W)
    l_aux = laux[0, 0]
    iv = idx_eff.reshape(S, 1, 1)
    lv = loc_i.reshape(S, 1, 1)
    dispatch = (iv == jnp.arange(E, dtype=jnp.int32).reshape(1, E, 1)) & \
               (lv == jnp.arange(C, dtype=jnp.int32).reshape(1, 1, C))
    return (l_aux, combine, dispatch)
